# baseline (speedup 1.0000x reference)
import functools

import jax
import jax.numpy as jnp
from jax import lax
from jax.experimental import pallas as pl
from jax.experimental.pallas import tpu as pltpu
from jax.experimental.pallas import tpu_sc as plsc

HIDDEN = 64
NUM_CORES = 2
NUM_SUBCORES = 16
NUM_WORKERS = NUM_CORES * NUM_SUBCORES
CHUNK = 128
NBUF = 2


def _gather_kernel(n_chunks, idx_hbm, table_hbm, out_hbm,
                   idx_v, rows_v, gsems, wsems):
  n_rounds = n_chunks // NBUF
  b_per_w = n_chunks * CHUNK
  wid = lax.axis_index("s") * NUM_CORES + lax.axis_index("c")
  wbase = wid * b_per_w

  pltpu.sync_copy(
      idx_hbm.at[pl.ds(pl.multiple_of(wid * n_chunks, 8), n_chunks)], idx_v)

  def gather_copy(g, s):
    return pltpu.make_async_copy(
        table_hbm.at[idx_v.at[g]],
        rows_v.at[s],
        gsems[s])

  def write_copy(g, s):
    return pltpu.make_async_copy(
        rows_v.at[s],
        out_hbm.at[pl.ds(pl.multiple_of(wbase + g * CHUNK, 8), CHUNK)],
        wsems[s])

  for s in range(NBUF):
    gather_copy(s, s).start()

  def body(r, carry):
    for s in range(NBUF):
      g = r * NBUF + s
      gather_copy(g, s).wait()
      write_copy(g, s).start()
    for s in range(NBUF):
      g = r * NBUF + s
      write_copy(g, s).wait()
      gather_copy(g + NBUF, s).start()
    return carry

  lax.fori_loop(0, n_rounds - 1, body, 0)

  r = n_rounds - 1
  for s in range(NBUF):
    g = r * NBUF + s
    gather_copy(g, s).wait()
    write_copy(g, s).start()
  for s in range(NBUF):
    write_copy(r * NBUF + s, s).wait()


def _build_call(n_idx):
  assert n_idx % (NUM_WORKERS * CHUNK * NBUF) == 0
  n_chunks = n_idx // (NUM_WORKERS * CHUNK)
  assert (n_chunks * NUM_WORKERS) % 8 == 0 and n_chunks % 8 == 0
  mesh = plsc.VectorSubcoreMesh(core_axis_name="c", subcore_axis_name="s")
  return pl.kernel(
      functools.partial(_gather_kernel, n_chunks),
      out_type=jax.ShapeDtypeStruct((n_idx, HIDDEN), jnp.float32),
      mesh=mesh,
      scratch_types=[
          pltpu.VMEM((n_chunks, CHUNK), jnp.int32),
          pltpu.VMEM((NBUF, CHUNK, HIDDEN), jnp.float32),
          [pltpu.SemaphoreType.DMA] * NBUF,
          [pltpu.SemaphoreType.DMA] * NBUF,
      ],
      compiler_params=pltpu.CompilerParams(use_tc_tiling_on_sc=False),
  )


@jax.jit
def kernel(input_ids, table):
  shape = input_ids.shape
  idx_flat = input_ids.reshape(-1, CHUNK).astype(jnp.int32)
  out = _build_call(idx_flat.size)(idx_flat, table)
  return out.reshape(shape + (HIDDEN,))


# speedup vs baseline: 1.0343x; 1.0343x over previous
import functools

import jax
import jax.numpy as jnp
from jax import lax
from jax.experimental import pallas as pl
from jax.experimental.pallas import tpu as pltpu
from jax.experimental.pallas import tpu_sc as plsc

HIDDEN = 64
NUM_CORES = 2
NUM_SUBCORES = 16
NUM_WORKERS = NUM_CORES * NUM_SUBCORES
CHUNK = 128
NBUF = 8


def _gather_kernel(n_chunks, idx_hbm, table_hbm, out_hbm,
                   idx_v, rows_v, gsems, wsems):
  n_rounds = n_chunks // NBUF
  b_per_w = n_chunks * CHUNK
  wid = lax.axis_index("s") * NUM_CORES + lax.axis_index("c")
  wbase = wid * b_per_w

  pltpu.sync_copy(
      idx_hbm.at[pl.ds(pl.multiple_of(wid * n_chunks, 8), n_chunks)], idx_v)

  def gather_copy(g, s):
    return pltpu.make_async_copy(
        table_hbm.at[idx_v.at[g]],
        rows_v.at[s],
        gsems[s])

  def write_copy(g, s):
    return pltpu.make_async_copy(
        rows_v.at[s],
        out_hbm.at[pl.ds(pl.multiple_of(wbase + g * CHUNK, 8), CHUNK)],
        wsems[s])

  for s in range(NBUF):
    gather_copy(s, s).start()

  def body(r, carry):
    for s in range(NBUF):
      g = r * NBUF + s
      gather_copy(g, s).wait()
      write_copy(g, s).start()
    for s in range(NBUF):
      g = r * NBUF + s
      write_copy(g, s).wait()
      gather_copy(g + NBUF, s).start()
    return carry

  lax.fori_loop(0, n_rounds - 1, body, 0)

  r = n_rounds - 1
  for s in range(NBUF):
    g = r * NBUF + s
    gather_copy(g, s).wait()
    write_copy(g, s).start()
  for s in range(NBUF):
    write_copy(r * NBUF + s, s).wait()


def _build_call(n_idx):
  assert n_idx % (NUM_WORKERS * CHUNK * NBUF) == 0
  n_chunks = n_idx // (NUM_WORKERS * CHUNK)
  assert (n_chunks * NUM_WORKERS) % 8 == 0 and n_chunks % 8 == 0
  mesh = plsc.VectorSubcoreMesh(core_axis_name="c", subcore_axis_name="s")
  return pl.kernel(
      functools.partial(_gather_kernel, n_chunks),
      out_type=jax.ShapeDtypeStruct((n_idx, HIDDEN), jnp.float32),
      mesh=mesh,
      scratch_types=[
          pltpu.VMEM((n_chunks, CHUNK), jnp.int32),
          pltpu.VMEM((NBUF, CHUNK, HIDDEN), jnp.float32),
          [pltpu.SemaphoreType.DMA] * NBUF,
          [pltpu.SemaphoreType.DMA] * NBUF,
      ],
      compiler_params=pltpu.CompilerParams(use_tc_tiling_on_sc=False),
  )


@jax.jit
def kernel(input_ids, table):
  shape = input_ids.shape
  idx_flat = input_ids.reshape(-1, CHUNK).astype(jnp.int32)
  out = _build_call(idx_flat.size)(idx_flat, table)
  return out.reshape(shape + (HIDDEN,))


# speedup vs baseline: 1.0356x; 1.0012x over previous
import functools

import jax
import jax.numpy as jnp
from jax import lax
from jax.experimental import layout as jax_layout
from jax.experimental import pallas as pl
from jax.experimental.pallas import tpu as pltpu
from jax.experimental.pallas import tpu_sc as plsc

HIDDEN = 64
NUM_CORES = 2
NUM_SUBCORES = 16
NUM_WORKERS = NUM_CORES * NUM_SUBCORES
CHUNK = 128
NBUF = 8


def _gather_kernel(n_chunks, idx_hbm, table_hbm, out_hbm,
                   idx_v, rows_v, gsems, wsems):
  n_rounds = n_chunks // NBUF
  b_per_w = n_chunks * CHUNK
  wid = lax.axis_index("s") * NUM_CORES + lax.axis_index("c")
  wbase = wid * b_per_w

  pltpu.sync_copy(
      idx_hbm.at[pl.ds(pl.multiple_of(wid * n_chunks, 8), n_chunks)], idx_v)

  def gather_copy(g, s):
    return pltpu.make_async_copy(
        table_hbm.at[idx_v.at[g]],
        rows_v.at[s],
        gsems[s])

  def write_copy(g, s):
    return pltpu.make_async_copy(
        rows_v.at[s],
        out_hbm.at[pl.ds(pl.multiple_of(wbase + g * CHUNK, 8), CHUNK)],
        wsems[s])

  for s in range(NBUF):
    gather_copy(s, s).start()

  def body(r, carry):
    for s in range(NBUF):
      g = r * NBUF + s
      gather_copy(g, s).wait()
      write_copy(g, s).start()
    for s in range(NBUF):
      g = r * NBUF + s
      write_copy(g, s).wait()
      gather_copy(g + NBUF, s).start()
    return carry

  lax.fori_loop(0, n_rounds - 1, body, 0)

  r = n_rounds - 1
  for s in range(NBUF):
    g = r * NBUF + s
    gather_copy(g, s).wait()
    write_copy(g, s).start()
  for s in range(NBUF):
    write_copy(r * NBUF + s, s).wait()


def _build_call(n_idx):
  assert n_idx % (NUM_WORKERS * CHUNK * NBUF) == 0
  n_chunks = n_idx // (NUM_WORKERS * CHUNK)
  assert (n_chunks * NUM_WORKERS) % 8 == 0 and n_chunks % 8 == 0
  mesh = plsc.VectorSubcoreMesh(core_axis_name="c", subcore_axis_name="s")
  return pl.kernel(
      functools.partial(_gather_kernel, n_chunks),
      out_type=jax.ShapeDtypeStruct((n_idx, HIDDEN), jnp.float32),
      mesh=mesh,
      scratch_types=[
          pltpu.VMEM((n_chunks, CHUNK), jnp.int32),
          pltpu.VMEM((NBUF, CHUNK, HIDDEN), jnp.float32),
          [pltpu.SemaphoreType.DMA] * NBUF,
          [pltpu.SemaphoreType.DMA] * NBUF,
      ],
      compiler_params=pltpu.CompilerParams(use_tc_tiling_on_sc=False),
  )


def _impl(input_ids, table):
  shape = input_ids.shape
  idx_flat = input_ids.reshape(-1, CHUNK).astype(jnp.int32)
  out = _build_call(idx_flat.size)(idx_flat, table)
  return out.reshape(shape + (HIDDEN,))


@functools.lru_cache(maxsize=None)
def _jitted(device):
  fmt = jax_layout.Format(
      jax_layout.Layout(major_to_minor=(0, 1, 2), tiling=((8,),)),
      jax.sharding.SingleDeviceSharding(device))
  return jax.jit(_impl, out_shardings=fmt)


def kernel(input_ids, table):
  try:
    device = next(iter(table.devices()))
  except Exception:
    device = jax.devices()[0]
  return _jitted(device)(input_ids, table)
